# baseline (speedup 1.0000x reference)
import functools

import jax
import jax.numpy as jnp
from jax import lax
from jax.experimental import pallas as pl
from jax.experimental.pallas import tpu as pltpu
from jax.experimental.pallas import tpu_sc as plsc

B, H, W = 4, 512, 512
HW = H * W
D3, D4 = 33, 16
C3 = D3 * D3 * D3
C3P = 35952
T3 = 3 * C3P
C4 = D4 ** 4

NC, NS = 2, 16
NW = NC * NS
CHUNKS = NW // B
PPW = HW // CHUNKS
BLK = 2048
NBLK = PPW // BLK
NV = BLK // 16


def _axis_prep(x, d):
    c = jnp.minimum(jnp.maximum(x, 0.0), 1.0) * (d - 1)
    i0 = jnp.minimum(c.astype(jnp.int32), d - 2)
    f = c - i0.astype(jnp.float32)
    return i0, f


@functools.cache
def _build_net_sc():
    mesh = plsc.VectorSubcoreMesh(
        core_axis_name="c", subcore_axis_name="s",
        num_cores=NC, num_subcores=NS)
    return pl.kernel(
        _net_sc_body,
        out_type=(jax.ShapeDtypeStruct((B * 3 * HW,), jnp.float32),
                  jax.ShapeDtypeStruct((B * 3 * HW,), jnp.float32)),
        mesh=mesh,
        scratch_types=[
            pltpu.VMEM((T3,), jnp.float32),
            pltpu.VMEM((4 * BLK,), jnp.float32),
            pltpu.VMEM((3 * BLK,), jnp.float32),
        ],
        compiler_params=pltpu.CompilerParams(needs_layout_passes=False),
    )


def _net_sc_body(vi, ir, lut4, l30, l31, l32, l33, lpgf, out, inter, table,
                 inbuf, outbuf):
    cid = lax.axis_index("c")
    sid = lax.axis_index("s")
    wid = sid * NC + cid
    b = wid // CHUNKS
    pbase = (wid % CHUNKS) * PPW
    ibase = wid * NBLK * 3 * BLK

    for ch in range(3):
        pltpu.sync_copy(lut4.at[pl.ds(ch * C4, C4)], table.at[pl.ds(0, C4)])

        def blk0(blk, _, ch=ch):
            off = pbase + blk * BLK
            for ax in range(3):
                pltpu.sync_copy(vi.at[pl.ds((b * 3 + ax) * HW + off, BLK)],
                                inbuf.at[pl.ds(ax * BLK, BLK)])
            pltpu.sync_copy(ir.at[pl.ds(b * HW + off, BLK)],
                            inbuf.at[pl.ds(3 * BLK, BLK)])

            @plsc.parallel_loop(0, NV, unroll=4)
            def vec0(v):
                i0, f = zip(*[_axis_prep(inbuf[pl.ds(ax * BLK + v * 16, 16)],
                                         D4) for ax in range(4)])
                base = ((i0[0] * D4 + i0[1]) * D4 + i0[2]) * D4 + i0[3]
                t0 = (1.0 - f[0], f[0])
                t1 = (1.0 - f[1], f[1])
                t2 = (1.0 - f[2], f[2])
                t3 = (1.0 - f[3], f[3])
                p01 = [[t0[b0] * t1[b1] for b0 in range(2)]
                       for b1 in range(2)]
                acc = None
                for bits in range(16):
                    b0, b1 = bits & 1, (bits >> 1) & 1
                    b2, b3 = (bits >> 2) & 1, (bits >> 3) & 1
                    co = b0 * D4 ** 3 + b1 * D4 ** 2 + b2 * D4 + b3
                    w = (p01[b1][b0] * t2[b2]) * t3[b3]
                    g = plsc.load_gather(table, [base + co])
                    t = g * w
                    acc = t if acc is None else acc + t
                outbuf[pl.ds(v * 16, 16)] = (
                    jnp.minimum(jnp.maximum(acc, 0.0), 1.0))

            pltpu.sync_copy(
                outbuf.at[pl.ds(0, BLK)],
                inter.at[pl.ds(ibase + (blk * 3 + ch) * BLK, BLK)])
            return _

        lax.fori_loop(0, NBLK, blk0, None)

    def trilinear_block(clip_out):
        @plsc.parallel_loop(0, NV, unroll=4)
        def vec3(v):
            i0, f = zip(*[_axis_prep(inbuf[pl.ds(ax * BLK + v * 16, 16)], D3)
                          for ax in range(3)])
            base = (i0[0] * D3 + i0[1]) * D3 + i0[2]
            wr = (1.0 - f[0], f[0])
            wg = (1.0 - f[1], f[1])
            wb = (1.0 - f[2], f[2])
            corners = []
            for rb in range(2):
                for gb in range(2):
                    wrg = wr[rb] * wg[gb]
                    for bb in range(2):
                        corners.append((rb * (D3 * D3) + gb * D3 + bb,
                                        wrg * wb[bb]))
            for ch in range(3):
                cbase = base + ch * C3P
                acc = None
                for co, w in corners:
                    g = plsc.load_gather(table, [cbase + co])
                    t = g * w
                    acc = t if acc is None else acc + t
                if clip_out:
                    acc = jnp.minimum(jnp.maximum(acc, 0.0), 1.0)
                outbuf[pl.ds(ch * BLK + v * 16, 16)] = acc

        return vec3

    for lref in (l30, l31, l32, l33):
        pltpu.sync_copy(lref.at[pl.ds(b * T3, T3)], table)

        def blk3(blk, _):
            ioff = ibase + blk * 3 * BLK
            pltpu.sync_copy(inter.at[pl.ds(ioff, 3 * BLK)],
                            inbuf.at[pl.ds(0, 3 * BLK)])
            trilinear_block(True)
            pltpu.sync_copy(outbuf, inter.at[pl.ds(ioff, 3 * BLK)])
            return _

        lax.fori_loop(0, NBLK, blk3, None)

    pltpu.sync_copy(lpgf, table)

    def blk5(blk, _):
        pltpu.sync_copy(inter.at[pl.ds(ibase + blk * 3 * BLK, 3 * BLK)],
                        inbuf.at[pl.ds(0, 3 * BLK)])
        trilinear_block(False)
        off = pbase + blk * BLK
        for ch in range(3):
            pltpu.sync_copy(outbuf.at[pl.ds(ch * BLK, BLK)],
                            out.at[pl.ds((b * 3 + ch) * HW + off, BLK)])
        return _

    lax.fori_loop(0, NBLK, blk5, None)


def _pad3(l):
    flat = l.reshape(l.shape[:-4] + (3, C3))
    pad = [(0, 0)] * (flat.ndim - 1) + [(0, C3P - C3)]
    return jnp.pad(flat, pad).reshape(l.shape[:-4] + (T3,))


@jax.jit
def kernel(vi_image, ir_image, lut4d, lut3d_00, lut3d_01, lut3d_02,
           lut3d_03, lut_pgf):
    vi = vi_image.reshape(B * 3 * HW)
    ir = ir_image.reshape(B * HW)
    l4 = lut4d.reshape(3 * C4)
    out, _ = _build_net_sc()(
        vi, ir, l4, _pad3(lut3d_00).reshape(B * T3),
        _pad3(lut3d_01).reshape(B * T3), _pad3(lut3d_02).reshape(B * T3),
        _pad3(lut3d_03).reshape(B * T3), _pad3(lut_pgf))
    return out.reshape(B, 3, H, W)


# speedup vs baseline: 1.4570x; 1.4570x over previous
import functools

import jax
import jax.numpy as jnp
from jax import lax
from jax.experimental import pallas as pl
from jax.experimental.pallas import tpu as pltpu
from jax.experimental.pallas import tpu_sc as plsc

B, H, W = 4, 512, 512
HW = H * W
D3, D4 = 33, 16
C3 = D3 * D3 * D3
C3P = 35952
T3 = 3 * C3P
C4 = D4 ** 4

NC, NS = 2, 16
NW = NC * NS
CHUNKS = NW // B
PPW = HW // CHUNKS
BLK = 1024
NBLK = PPW // BLK
NV = BLK // 16
ISTR = 4 * BLK
OSTR = 3 * BLK


def _axis_prep(x, d):
    c = jnp.minimum(jnp.maximum(x, 0.0), 1.0) * (d - 1)
    i0 = jnp.minimum(c.astype(jnp.int32), d - 2)
    f = c - i0.astype(jnp.float32)
    return i0, f


@functools.cache
def _build_net_sc():
    mesh = plsc.VectorSubcoreMesh(
        core_axis_name="c", subcore_axis_name="s",
        num_cores=NC, num_subcores=NS)
    return pl.kernel(
        _net_sc_body,
        out_type=(jax.ShapeDtypeStruct((B * 3 * HW,), jnp.float32),
                  jax.ShapeDtypeStruct((B * 3 * HW,), jnp.float32)),
        mesh=mesh,
        scratch_types=[
            pltpu.VMEM((T3,), jnp.float32),
            pltpu.VMEM((2 * ISTR,), jnp.float32),
            pltpu.VMEM((2 * OSTR,), jnp.float32),
            pltpu.SemaphoreType.DMA,
            pltpu.SemaphoreType.DMA,
            pltpu.SemaphoreType.DMA,
            pltpu.SemaphoreType.DMA,
        ],
        compiler_params=pltpu.CompilerParams(needs_layout_passes=False),
    )


def _run_stage(issue_in, wait_in, compute, issue_out, wait_out):
    issue_in(0, 0)

    def pair(i, _):
        blk0, blk1 = 2 * i, 2 * i + 1
        issue_in(blk1, 1)
        wait_in(0)

        @pl.when(i > 0)
        def _():
            wait_out(0)

        compute(0)
        issue_out(blk0, 0)
        issue_in(jnp.minimum(blk1 + 1, NBLK - 1), 0)
        wait_in(1)

        @pl.when(i > 0)
        def _():
            wait_out(1)

        compute(1)
        issue_out(blk1, 1)
        return _

    lax.fori_loop(0, NBLK // 2, pair, None)
    wait_in(0)
    wait_out(0)
    wait_out(1)


def _net_sc_body(vi, ir, lut4, l30, l31, l32, l33, lpgf, out, inter, table,
                 inbuf, outbuf, semi0, semi1, semo0, semo1):
    cid = lax.axis_index("c")
    sid = lax.axis_index("s")
    wid = sid * NC + cid
    b = wid // CHUNKS
    pbase = (wid % CHUNKS) * PPW
    ibase = wid * NBLK * 3 * BLK
    semi = (semi0, semi1)
    semo = (semo0, semo1)

    def issue_in0(blk, slot):
        off = pbase + blk * BLK
        for ax in range(3):
            pltpu.async_copy(vi.at[pl.ds((b * 3 + ax) * HW + off, BLK)],
                             inbuf.at[pl.ds(slot * ISTR + ax * BLK, BLK)],
                             semi[slot])
        pltpu.async_copy(ir.at[pl.ds(b * HW + off, BLK)],
                         inbuf.at[pl.ds(slot * ISTR + 3 * BLK, BLK)],
                         semi[slot])

    def wait_in0(slot):
        pltpu.make_async_copy(vi.at[pl.ds(0, ISTR)],
                              inbuf.at[pl.ds(slot * ISTR, ISTR)],
                              semi[slot]).wait()

    def compute0(slot):
        @plsc.parallel_loop(0, NV, unroll=2)
        def vec0(v):
            i0, f = zip(*[
                _axis_prep(inbuf[pl.ds(slot * ISTR + ax * BLK + v * 16, 16)],
                           D4) for ax in range(4)])
            base = ((i0[0] * D4 + i0[1]) * D4 + i0[2]) * D4 + i0[3]
            t0 = (1.0 - f[0], f[0])
            t1 = (1.0 - f[1], f[1])
            t2 = (1.0 - f[2], f[2])
            t3 = (1.0 - f[3], f[3])
            p01 = [[t0[b0] * t1[b1] for b0 in range(2)] for b1 in range(2)]
            acc = None
            for bits in range(16):
                b0, b1 = bits & 1, (bits >> 1) & 1
                b2, b3 = (bits >> 2) & 1, (bits >> 3) & 1
                co = b0 * D4 ** 3 + b1 * D4 ** 2 + b2 * D4 + b3
                w = (p01[b1][b0] * t2[b2]) * t3[b3]
                g = plsc.load_gather(table, [base + co])
                t = g * w
                acc = t if acc is None else acc + t
            outbuf[pl.ds(slot * OSTR + v * 16, 16)] = (
                jnp.minimum(jnp.maximum(acc, 0.0), 1.0))

    for ch in range(3):
        pltpu.sync_copy(lut4.at[pl.ds(ch * C4, C4)], table.at[pl.ds(0, C4)])

        def issue_out0(blk, slot, ch=ch):
            pltpu.async_copy(
                outbuf.at[pl.ds(slot * OSTR, BLK)],
                inter.at[pl.ds(ibase + (blk * 3 + ch) * BLK, BLK)],
                semo[slot])

        def wait_out0(slot):
            pltpu.make_async_copy(outbuf.at[pl.ds(slot * OSTR, BLK)],
                                  inter.at[pl.ds(ibase, BLK)],
                                  semo[slot]).wait()

        _run_stage(issue_in0, wait_in0, compute0, issue_out0, wait_out0)

    def issue_in3(blk, slot):
        pltpu.async_copy(inter.at[pl.ds(ibase + blk * 3 * BLK, 3 * BLK)],
                         inbuf.at[pl.ds(slot * ISTR, 3 * BLK)],
                         semi[slot])

    def wait_in3(slot):
        pltpu.make_async_copy(inter.at[pl.ds(ibase, 3 * BLK)],
                              inbuf.at[pl.ds(slot * ISTR, 3 * BLK)],
                              semi[slot]).wait()

    def compute3(slot, clip_out):
        @plsc.parallel_loop(0, NV, unroll=2)
        def vec3(v):
            i0, f = zip(*[
                _axis_prep(inbuf[pl.ds(slot * ISTR + ax * BLK + v * 16, 16)],
                           D3) for ax in range(3)])
            base = (i0[0] * D3 + i0[1]) * D3 + i0[2]
            wr = (1.0 - f[0], f[0])
            wg = (1.0 - f[1], f[1])
            wb = (1.0 - f[2], f[2])
            corners = []
            for rb in range(2):
                for gb in range(2):
                    wrg = wr[rb] * wg[gb]
                    for bb in range(2):
                        corners.append((rb * (D3 * D3) + gb * D3 + bb,
                                        wrg * wb[bb]))
            for ch in range(3):
                cbase = base + ch * C3P
                acc = None
                for co, w in corners:
                    g = plsc.load_gather(table, [cbase + co])
                    t = g * w
                    acc = t if acc is None else acc + t
                if clip_out:
                    acc = jnp.minimum(jnp.maximum(acc, 0.0), 1.0)
                outbuf[pl.ds(slot * OSTR + ch * BLK + v * 16, 16)] = acc

    def issue_out3(blk, slot):
        pltpu.async_copy(outbuf.at[pl.ds(slot * OSTR, 3 * BLK)],
                         inter.at[pl.ds(ibase + blk * 3 * BLK, 3 * BLK)],
                         semo[slot])

    def wait_out3(slot):
        pltpu.make_async_copy(outbuf.at[pl.ds(slot * OSTR, 3 * BLK)],
                              inter.at[pl.ds(ibase, 3 * BLK)],
                              semo[slot]).wait()

    for lref in (l30, l31, l32, l33):
        pltpu.sync_copy(lref.at[pl.ds(b * T3, T3)], table)
        _run_stage(issue_in3, wait_in3,
                   lambda slot: compute3(slot, True), issue_out3, wait_out3)

    pltpu.sync_copy(lpgf, table)

    def issue_out5(blk, slot):
        off = pbase + blk * BLK
        for ch in range(3):
            pltpu.async_copy(
                outbuf.at[pl.ds(slot * OSTR + ch * BLK, BLK)],
                out.at[pl.ds((b * 3 + ch) * HW + off, BLK)],
                semo[slot])

    def wait_out5(slot):
        pltpu.make_async_copy(outbuf.at[pl.ds(slot * OSTR, 3 * BLK)],
                              out.at[pl.ds(0, 3 * BLK)],
                              semo[slot]).wait()

    _run_stage(issue_in3, wait_in3,
               lambda slot: compute3(slot, False), issue_out5, wait_out5)


def _pad3(l):
    flat = l.reshape(l.shape[:-4] + (3, C3))
    pad = [(0, 0)] * (flat.ndim - 1) + [(0, C3P - C3)]
    return jnp.pad(flat, pad).reshape(l.shape[:-4] + (T3,))


@jax.jit
def kernel(vi_image, ir_image, lut4d, lut3d_00, lut3d_01, lut3d_02,
           lut3d_03, lut_pgf):
    vi = vi_image.reshape(B * 3 * HW)
    ir = ir_image.reshape(B * HW)
    l4 = lut4d.reshape(3 * C4)
    out, _ = _build_net_sc()(
        vi, ir, l4, _pad3(lut3d_00).reshape(B * T3),
        _pad3(lut3d_01).reshape(B * T3), _pad3(lut3d_02).reshape(B * T3),
        _pad3(lut3d_03).reshape(B * T3), _pad3(lut_pgf))
    return out.reshape(B, 3, H, W)


# speedup vs baseline: 1.5544x; 1.0669x over previous
import functools

import jax
import jax.numpy as jnp
from jax import lax
from jax.experimental import pallas as pl
from jax.experimental.pallas import tpu as pltpu
from jax.experimental.pallas import tpu_sc as plsc

B, H, W = 4, 512, 512
HW = H * W
D3, D4 = 33, 16
C3 = D3 * D3 * D3
C3P = 35952
T3 = 3 * C3P
C4 = D4 ** 4

NC, NS = 2, 16
NW = NC * NS
CHUNKS = NW // B
PPW = HW // CHUNKS
BLK = 1024
NBLK = PPW // BLK
NV = BLK // 16
ISTR = 4 * BLK
OSTR = 3 * BLK


def _axis_prep(x, d):
    c = jnp.minimum(jnp.maximum(x, 0.0), 1.0) * (d - 1)
    i0 = jnp.minimum(c.astype(jnp.int32), d - 2)
    f = c - i0.astype(jnp.float32)
    return i0, f


def _axis_prep_preclipped(x, d):
    c = x * (d - 1)
    i0 = jnp.minimum(c.astype(jnp.int32), d - 2)
    f = c - i0.astype(jnp.float32)
    return i0, f


@functools.cache
def _build_net_sc():
    mesh = plsc.VectorSubcoreMesh(
        core_axis_name="c", subcore_axis_name="s",
        num_cores=NC, num_subcores=NS)
    return pl.kernel(
        _net_sc_body,
        out_type=(jax.ShapeDtypeStruct((B * 3 * HW,), jnp.float32),
                  jax.ShapeDtypeStruct((B * 3 * HW,), jnp.float32)),
        mesh=mesh,
        scratch_types=[
            pltpu.VMEM((T3,), jnp.float32),
            pltpu.VMEM((2 * ISTR,), jnp.float32),
            pltpu.VMEM((2 * OSTR,), jnp.float32),
            pltpu.SemaphoreType.DMA,
            pltpu.SemaphoreType.DMA,
            pltpu.SemaphoreType.DMA,
            pltpu.SemaphoreType.DMA,
        ],
        compiler_params=pltpu.CompilerParams(needs_layout_passes=False),
    )


def _run_stage(issue_in, wait_in, compute, issue_out, wait_out):
    issue_in(0, 0)

    def pair(i, _):
        blk0, blk1 = 2 * i, 2 * i + 1
        issue_in(blk1, 1)
        wait_in(0)

        @pl.when(i > 0)
        def _():
            wait_out(0)

        compute(0)
        issue_out(blk0, 0)
        issue_in(jnp.minimum(blk1 + 1, NBLK - 1), 0)
        wait_in(1)

        @pl.when(i > 0)
        def _():
            wait_out(1)

        compute(1)
        issue_out(blk1, 1)
        return _

    lax.fori_loop(0, NBLK // 2, pair, None)
    wait_in(0)
    wait_out(0)
    wait_out(1)


def _net_sc_body(vi, ir, lut4, l30, l31, l32, l33, lpgf, out, inter, table,
                 inbuf, outbuf, semi0, semi1, semo0, semo1):
    cid = lax.axis_index("c")
    sid = lax.axis_index("s")
    wid = sid * NC + cid
    b = wid // CHUNKS
    pbase = (wid % CHUNKS) * PPW
    ibase = wid * NBLK * 3 * BLK
    semi = (semi0, semi1)
    semo = (semo0, semo1)

    def issue_in0(blk, slot):
        off = pbase + blk * BLK
        for ax in range(3):
            pltpu.async_copy(vi.at[pl.ds((b * 3 + ax) * HW + off, BLK)],
                             inbuf.at[pl.ds(slot * ISTR + ax * BLK, BLK)],
                             semi[slot])
        pltpu.async_copy(ir.at[pl.ds(b * HW + off, BLK)],
                         inbuf.at[pl.ds(slot * ISTR + 3 * BLK, BLK)],
                         semi[slot])

    def wait_in0(slot):
        pltpu.make_async_copy(vi.at[pl.ds(0, ISTR)],
                              inbuf.at[pl.ds(slot * ISTR, ISTR)],
                              semi[slot]).wait()

    def compute0(slot):
        @plsc.parallel_loop(0, NV, unroll=2)
        def vec0(v):
            i0, f = zip(*[
                _axis_prep(inbuf[pl.ds(slot * ISTR + ax * BLK + v * 16, 16)],
                           D4) for ax in range(4)])
            base = ((i0[0] * D4 + i0[1]) * D4 + i0[2]) * D4 + i0[3]
            t0 = (1.0 - f[0], f[0])
            t1 = (1.0 - f[1], f[1])
            t2 = (1.0 - f[2], f[2])
            t3 = (1.0 - f[3], f[3])
            p01 = [[t0[b0] * t1[b1] for b0 in range(2)] for b1 in range(2)]
            acc = None
            for bits in range(16):
                b0, b1 = bits & 1, (bits >> 1) & 1
                b2, b3 = (bits >> 2) & 1, (bits >> 3) & 1
                co = b0 * D4 ** 3 + b1 * D4 ** 2 + b2 * D4 + b3
                w = (p01[b1][b0] * t2[b2]) * t3[b3]
                g = plsc.load_gather(table, [base + co])
                t = g * w
                acc = t if acc is None else acc + t
            outbuf[pl.ds(slot * OSTR + v * 16, 16)] = (
                jnp.minimum(jnp.maximum(acc, 0.0), 1.0))

    for ch in range(3):
        pltpu.sync_copy(lut4.at[pl.ds(ch * C4, C4)], table.at[pl.ds(0, C4)])

        def issue_out0(blk, slot, ch=ch):
            pltpu.async_copy(
                outbuf.at[pl.ds(slot * OSTR, BLK)],
                inter.at[pl.ds(ibase + (blk * 3 + ch) * BLK, BLK)],
                semo[slot])

        def wait_out0(slot):
            pltpu.make_async_copy(outbuf.at[pl.ds(slot * OSTR, BLK)],
                                  inter.at[pl.ds(ibase, BLK)],
                                  semo[slot]).wait()

        _run_stage(issue_in0, wait_in0, compute0, issue_out0, wait_out0)

    def issue_in3(blk, slot):
        pltpu.async_copy(inter.at[pl.ds(ibase + blk * 3 * BLK, 3 * BLK)],
                         inbuf.at[pl.ds(slot * ISTR, 3 * BLK)],
                         semi[slot])

    def wait_in3(slot):
        pltpu.make_async_copy(inter.at[pl.ds(ibase, 3 * BLK)],
                              inbuf.at[pl.ds(slot * ISTR, 3 * BLK)],
                              semi[slot]).wait()

    def compute3(slot, clip_out):
        @plsc.parallel_loop(0, NV, unroll=2)
        def vec3(v):
            i0, f = zip(*[
                _axis_prep_preclipped(
                    inbuf[pl.ds(slot * ISTR + ax * BLK + v * 16, 16)], D3)
                for ax in range(3)])
            base = (i0[0] * D3 + i0[1]) * D3 + i0[2]
            wr = (1.0 - f[0], f[0])
            wg = (1.0 - f[1], f[1])
            wb = (1.0 - f[2], f[2])
            corners = []
            for rb in range(2):
                for gb in range(2):
                    wrg = wr[rb] * wg[gb]
                    for bb in range(2):
                        corners.append((rb * (D3 * D3) + gb * D3 + bb,
                                        wrg * wb[bb]))
            for ch in range(3):
                cbase = base + ch * C3P
                acc = None
                for co, w in corners:
                    g = plsc.load_gather(table, [cbase + co])
                    t = g * w
                    acc = t if acc is None else acc + t
                if clip_out:
                    acc = jnp.minimum(jnp.maximum(acc, 0.0), 1.0)
                outbuf[pl.ds(slot * OSTR + ch * BLK + v * 16, 16)] = acc

    def issue_out3(blk, slot):
        pltpu.async_copy(outbuf.at[pl.ds(slot * OSTR, 3 * BLK)],
                         inter.at[pl.ds(ibase + blk * 3 * BLK, 3 * BLK)],
                         semo[slot])

    def wait_out3(slot):
        pltpu.make_async_copy(outbuf.at[pl.ds(slot * OSTR, 3 * BLK)],
                              inter.at[pl.ds(ibase, 3 * BLK)],
                              semo[slot]).wait()

    for lref in (l30, l31, l32, l33):
        pltpu.sync_copy(lref.at[pl.ds(b * T3, T3)], table)
        _run_stage(issue_in3, wait_in3,
                   lambda slot: compute3(slot, True), issue_out3, wait_out3)

    pltpu.sync_copy(lpgf, table)

    def issue_out5(blk, slot):
        off = pbase + blk * BLK
        for ch in range(3):
            pltpu.async_copy(
                outbuf.at[pl.ds(slot * OSTR + ch * BLK, BLK)],
                out.at[pl.ds((b * 3 + ch) * HW + off, BLK)],
                semo[slot])

    def wait_out5(slot):
        pltpu.make_async_copy(outbuf.at[pl.ds(slot * OSTR, 3 * BLK)],
                              out.at[pl.ds(0, 3 * BLK)],
                              semo[slot]).wait()

    _run_stage(issue_in3, wait_in3,
               lambda slot: compute3(slot, False), issue_out5, wait_out5)


def _pad3(l):
    flat = l.reshape(l.shape[:-4] + (3, C3))
    pad = [(0, 0)] * (flat.ndim - 1) + [(0, C3P - C3)]
    return jnp.pad(flat, pad).reshape(l.shape[:-4] + (T3,))


@jax.jit
def kernel(vi_image, ir_image, lut4d, lut3d_00, lut3d_01, lut3d_02,
           lut3d_03, lut_pgf):
    vi = vi_image.reshape(B * 3 * HW)
    ir = ir_image.reshape(B * HW)
    l4 = lut4d.reshape(3 * C4)
    out, _ = _build_net_sc()(
        vi, ir, l4, _pad3(lut3d_00).reshape(B * T3),
        _pad3(lut3d_01).reshape(B * T3), _pad3(lut3d_02).reshape(B * T3),
        _pad3(lut3d_03).reshape(B * T3), _pad3(lut_pgf))
    return out.reshape(B, 3, H, W)
